# baseline (speedup 1.0000x reference)
import functools

import jax
import jax.numpy as jnp
from jax import lax
from jax.experimental import pallas as pl
from jax.experimental.pallas import tpu as pltpu
from jax.experimental.pallas import tpu_sc as plsc

B, N, C = 8, 5000, 8
F = 128
K = 64
CLS = 104

G = 2
BG = B // G

NC, NS = 2, 16
NW = NC * NS
NODES = BG * N
NPW = NODES // NW
NB = 5
NBLK = NPW // NB


NCH = 1
NT = N // NCH


def _prep_kernel(nodes_ref, ch_ref, w_ref, t_ref, rl_ref, coef_ref, gidx_ref):
    b = pl.program_id(0)
    t = pl.program_id(1)
    x = nodes_ref[0]
    ch = ch_ref[0]
    y = jnp.dot(x, w_ref[...], preferred_element_type=jnp.float32)
    t_ref[0] = y[:, :K]
    grow = lax.broadcasted_iota(jnp.int32, (NT, 1), 0) + t * NT
    rl_ref[0] = y[:, K:] * (grow != 0).astype(jnp.float32)

    cf = ch.astype(jnp.float32)
    m = jnp.minimum(cf, 1.0)
    nsib = jnp.sum((cf != 0.0).astype(jnp.float32), axis=1, keepdims=True)
    jj = lax.broadcasted_iota(jnp.int32, (NT, C), 1).astype(jnp.float32)
    denom = jnp.where(nsib == 1.0, 1.0, nsib - 1.0)
    denom = jnp.where(denom == 0.0, 1.0, denom)
    singles = jnp.where(jj == 0.0, 0.5, 0.0) + jnp.zeros_like(m)
    cr = jnp.where(nsib == 1.0, singles, jj * m / denom)
    cl = (1.0 - cr) * m
    coef_ref[0] = jnp.concatenate([cr, cl], axis=1)
    gidx_ref[0] = ch + b * N


def _prep_call(nodes, ch, w1, goff):
    return pl.pallas_call(
        _prep_kernel,
        grid=(BG, NCH),
        in_specs=[
            pl.BlockSpec((1, NT, F), lambda b, t: (b + goff, t, 0)),
            pl.BlockSpec((1, NT, C), lambda b, t: (b + goff, t, 0)),
            pl.BlockSpec((F, 3 * K), lambda b, t: (0, 0)),
        ],
        out_specs=[
            pl.BlockSpec((1, NT, K), lambda b, t: (b, t, 0)),
            pl.BlockSpec((1, NT, 2 * K), lambda b, t: (b, t, 0)),
            pl.BlockSpec((1, NT, 16), lambda b, t: (b, t, 0)),
            pl.BlockSpec((1, NT, C), lambda b, t: (b, t, 0)),
        ],
        out_shape=[
            jax.ShapeDtypeStruct((BG, N, K), jnp.float32),
            jax.ShapeDtypeStruct((BG, N, 2 * K), jnp.float32),
            jax.ShapeDtypeStruct((BG, N, 16), jnp.float32),
            jax.ShapeDtypeStruct((BG, N, C), jnp.int32),
        ],
    )(nodes, ch, w1)


_sc_mesh = plsc.VectorSubcoreMesh(core_axis_name="c", subcore_axis_name="s")


RING = 5


@functools.partial(
    pl.kernel,
    out_type=jax.ShapeDtypeStruct((NODES * K,), jnp.float32),
    mesh=_sc_mesh,
    scratch_types=[
        pltpu.VMEM((NPW * C,), jnp.int32),
        pltpu.VMEM((NPW * 16,), jnp.float32),
    ] + [pltpu.VMEM((NB * C, 2 * K), jnp.float32)] * RING
      + [pltpu.VMEM((NB * K,), jnp.float32)] * RING
      + [pltpu.SemaphoreType.DMA] * (2 * RING),
)
def _sc_mix(rl_hbm, gidx_hbm, coef_hbm, out_hbm, idx_all, coef_all, *rest):
    rows_v = rest[:RING]
    out_v = rest[RING:2 * RING]
    gsem = rest[2 * RING:3 * RING]
    osem = rest[3 * RING:]
    wid = lax.axis_index("s") * NC + lax.axis_index("c")
    wbase = wid * NPW

    pltpu.sync_copy(gidx_hbm.at[pl.ds(wbase * C, NPW * C)], idx_all)

    def gather_desc(slot, blk):
        isl = idx_all.at[pl.ds(blk * NB * C, NB * C)]
        return pltpu.make_async_copy(rl_hbm.at[isl], rows_v[slot], gsem[slot])

    def out_desc(slot, blk):
        dst = out_hbm.at[pl.ds((wbase + blk * NB) * K, NB * K)]
        return pltpu.make_async_copy(out_v[slot], dst, osem[slot])

    for r in range(RING):
        gather_desc(r, r).start()
    pltpu.sync_copy(coef_hbm.at[pl.ds(wbase * 16, NPW * 16)], coef_all)

    def group_body(gi, carry):
        for r in range(RING):
            blk = gi * RING + r
            gather_desc(r, blk).wait()

            @pl.when(gi > 0)
            def _():
                out_desc(r, blk - RING).wait()

            def node_body(n):
                cv = coef_all[pl.ds((blk * NB + n) * 16, 16)]
                ws = [cv.at[jnp.full((16,), i, jnp.int32)].get(
                    mode="promise_in_bounds") for i in range(16)]
                rr = rows_v[r]
                nc = n * C
                for k in range(4):
                    p = [ws[c] * rr[nc + c, pl.ds(k * 16, 16)]
                         + ws[8 + c] * rr[nc + c, pl.ds(K + k * 16, 16)]
                         for c in range(C)]
                    s = [p[0] + p[1], p[2] + p[3], p[4] + p[5], p[6] + p[7]]
                    out_v[r][pl.ds(n * K + k * 16, 16)] = (s[0] + s[1]) + (s[2] + s[3])

            for n in range(NB):
                node_body(n)
            out_desc(r, blk).start()

            @pl.when(blk + RING < NBLK)
            def _():
                gather_desc(r, blk + RING).start()
        return carry

    lax.fori_loop(0, NBLK // RING, group_body, 0)
    for r in range(RING):
        out_desc(r, NBLK - RING + r).wait()


def _mid_kernel(t_ref, mix_ref, b1_ref, w2_ref, t2_ref, rl2_ref):
    t = pl.program_id(1)
    pre = t_ref[0] + mix_ref[0] + b1_ref[...]
    x = jnp.tanh(pre)
    y = jnp.dot(x, w2_ref[...], preferred_element_type=jnp.float32)
    t2_ref[0] = y[:, :K]
    grow = lax.broadcasted_iota(jnp.int32, (NT, 1), 0) + t * NT
    rl2_ref[0] = y[:, K:] * (grow != 0).astype(jnp.float32)


def _mid_call(t1, mix1, b1, w2):
    return pl.pallas_call(
        _mid_kernel,
        grid=(BG, NCH),
        in_specs=[
            pl.BlockSpec((1, NT, K), lambda b, t: (b, t, 0)),
            pl.BlockSpec((1, NT, K), lambda b, t: (b, t, 0)),
            pl.BlockSpec((1, K), lambda b, t: (0, 0)),
            pl.BlockSpec((K, 3 * K), lambda b, t: (0, 0)),
        ],
        out_specs=[
            pl.BlockSpec((1, NT, K), lambda b, t: (b, t, 0)),
            pl.BlockSpec((1, NT, 2 * K), lambda b, t: (b, t, 0)),
        ],
        out_shape=[
            jax.ShapeDtypeStruct((BG, N, K), jnp.float32),
            jax.ShapeDtypeStruct((BG, N, 2 * K), jnp.float32),
        ],
    )(t1, mix1, b1, w2)


def _fin_kernel(t_ref, mix_ref, b2_ref, wc_ref, bc_ref, out_ref, acc_ref):
    t = pl.program_id(1)
    pre = t_ref[0] + mix_ref[0] + b2_ref[...]
    x = jnp.tanh(pre)
    part = jnp.max(x, axis=0, keepdims=True)

    @pl.when(t == 0)
    def _():
        acc_ref[...] = part

    @pl.when(t > 0)
    def _():
        acc_ref[...] = jnp.maximum(acc_ref[...], part)

    @pl.when(t == NCH - 1)
    def _():
        logits = jnp.dot(acc_ref[...], wc_ref[...],
                         preferred_element_type=jnp.float32) + bc_ref[...]
        mx = jnp.max(logits, axis=1, keepdims=True)
        e = jnp.exp(logits - mx)
        out_ref[0] = e / jnp.sum(e, axis=1, keepdims=True)


def _fin_call(t2, mix2, b2, w_cls, b_cls):
    return pl.pallas_call(
        _fin_kernel,
        grid=(BG, NCH),
        in_specs=[
            pl.BlockSpec((1, NT, K), lambda b, t: (b, t, 0)),
            pl.BlockSpec((1, NT, K), lambda b, t: (b, t, 0)),
            pl.BlockSpec((1, K), lambda b, t: (0, 0)),
            pl.BlockSpec((K, CLS), lambda b, t: (0, 0)),
            pl.BlockSpec((1, CLS), lambda b, t: (0, 0)),
        ],
        out_specs=pl.BlockSpec((1, 1, CLS), lambda b, t: (b, 0, 0)),
        out_shape=jax.ShapeDtypeStruct((BG, 1, CLS), jnp.float32),
        scratch_shapes=[pltpu.VMEM((1, K), jnp.float32)],
    )(t2, mix2, b2, w_cls, b_cls)


def kernel(nodes, children, w_t1, w_l1, w_r1, b1, w_t2, w_l2, w_r2, b2, w_cls, b_cls):
    ch = children.astype(jnp.int32)
    w1 = jnp.concatenate([w_t1, w_r1, w_l1], axis=1)
    w2 = jnp.concatenate([w_t2, w_r2, w_l2], axis=1)

    outs = []
    for g in range(G):
        t1, rl1, coef, gidx = _prep_call(nodes, ch, w1, g * BG)
        gidx_flat = gidx.reshape(NODES * C)
        coef_flat = coef.reshape(NODES * 16)
        mix1 = _sc_mix(rl1.reshape(NODES, 2 * K), gidx_flat, coef_flat)
        t2, rl2 = _mid_call(t1, mix1.reshape(BG, N, K), b1.reshape(1, K), w2)
        mix2 = _sc_mix(rl2.reshape(NODES, 2 * K), gidx_flat, coef_flat)
        outs.append(_fin_call(t2, mix2.reshape(BG, N, K), b2.reshape(1, K),
                              w_cls, b_cls.reshape(1, CLS)).reshape(BG, CLS))
    return jnp.concatenate(outs, axis=0)


# speedup vs baseline: 1.1786x; 1.1786x over previous
import functools

import jax
import jax.numpy as jnp
from jax import lax
from jax.experimental import pallas as pl
from jax.experimental.pallas import tpu as pltpu
from jax.experimental.pallas import tpu_sc as plsc

B, N, C = 8, 5000, 8
F = 128
K = 64
CLS = 104

G = 2
BG = B // G

NC, NS = 2, 16
NW = NC * NS
NODES = BG * N
NPW = NODES // NW
NB = 5
NBLK = NPW // NB


NCH = 1
NT = N // NCH


def _prep_kernel(nodes_ref, ch_ref, w_ref, t_ref, rl_ref, coef_ref, gidx_ref):
    b = pl.program_id(0)
    t = pl.program_id(1)
    x = nodes_ref[0]
    ch = ch_ref[0]
    y = jnp.dot(x, w_ref[...], preferred_element_type=jnp.float32)
    t_ref[0] = y[:, :K]
    grow = lax.broadcasted_iota(jnp.int32, (NT, 1), 0) + t * NT
    rl_ref[0] = y[:, K:] * (grow != 0).astype(jnp.float32)

    cf = ch.astype(jnp.float32)
    m = jnp.minimum(cf, 1.0)
    nsib = jnp.sum((cf != 0.0).astype(jnp.float32), axis=1, keepdims=True)
    jj = lax.broadcasted_iota(jnp.int32, (NT, C), 1).astype(jnp.float32)
    denom = jnp.where(nsib == 1.0, 1.0, nsib - 1.0)
    denom = jnp.where(denom == 0.0, 1.0, denom)
    singles = jnp.where(jj == 0.0, 0.5, 0.0) + jnp.zeros_like(m)
    cr = jnp.where(nsib == 1.0, singles, jj * m / denom)
    cl = (1.0 - cr) * m
    coef_ref[0] = jnp.concatenate([cr, cl], axis=1)
    gidx_ref[0] = ch + b * N


def _prep_call(nodes, ch, w1, goff):
    return pl.pallas_call(
        _prep_kernel,
        grid=(BG, NCH),
        in_specs=[
            pl.BlockSpec((1, NT, F), lambda b, t: (b + goff, t, 0)),
            pl.BlockSpec((1, NT, C), lambda b, t: (b + goff, t, 0)),
            pl.BlockSpec((F, 3 * K), lambda b, t: (0, 0)),
        ],
        out_specs=[
            pl.BlockSpec((1, NT, K), lambda b, t: (b, t, 0)),
            pl.BlockSpec((1, NT, 2 * K), lambda b, t: (b, t, 0)),
            pl.BlockSpec((1, NT, 16), lambda b, t: (b, t, 0)),
            pl.BlockSpec((1, NT, C), lambda b, t: (b, t, 0)),
        ],
        out_shape=[
            jax.ShapeDtypeStruct((BG, N, K), jnp.float32),
            jax.ShapeDtypeStruct((BG, N, 2 * K), jnp.float32),
            jax.ShapeDtypeStruct((BG, N, 16), jnp.float32),
            jax.ShapeDtypeStruct((BG, N, C), jnp.int32),
        ],
    )(nodes, ch, w1)


_sc_mesh = plsc.VectorSubcoreMesh(core_axis_name="c", subcore_axis_name="s")


RING = 5


@functools.partial(
    pl.kernel,
    out_type=jax.ShapeDtypeStruct((NODES * K,), jnp.float32),
    mesh=_sc_mesh,
    scratch_types=[
        pltpu.VMEM((NPW * C,), jnp.int32),
        pltpu.VMEM((NPW * 16,), jnp.float32),
    ] + [pltpu.VMEM((NB * C, 2 * K), jnp.float32)] * RING
      + [pltpu.VMEM((NB * K,), jnp.float32)] * RING
      + [pltpu.SemaphoreType.DMA] * (2 * RING),
)
def _sc_mix(rl_hbm, gidx_hbm, coef_hbm, out_hbm, idx_all, coef_all, *rest):
    rows_v = rest[:RING]
    out_v = rest[RING:2 * RING]
    gsem = rest[2 * RING:3 * RING]
    osem = rest[3 * RING:]
    wid = lax.axis_index("s") * NC + lax.axis_index("c")
    wbase = wid * NPW

    pltpu.sync_copy(gidx_hbm.at[pl.ds(wbase * C, NPW * C)], idx_all)

    def gather_desc(slot, blk):
        isl = idx_all.at[pl.ds(blk * NB * C, NB * C)]
        return pltpu.make_async_copy(rl_hbm.at[isl], rows_v[slot], gsem[slot])

    def out_desc(slot, blk):
        dst = out_hbm.at[pl.ds((wbase + blk * NB) * K, NB * K)]
        return pltpu.make_async_copy(out_v[slot], dst, osem[slot])

    for r in range(RING):
        gather_desc(r, r).start()
    pltpu.sync_copy(coef_hbm.at[pl.ds(wbase * 16, NPW * 16)], coef_all)

    def group_body(gi, carry):
        for r in range(RING):
            blk = gi * RING + r
            gather_desc(r, blk).wait()

            @pl.when(gi > 0)
            def _():
                out_desc(r, blk - RING).wait()

            def node_body(n, carry2):
                cv = coef_all[pl.ds((blk * NB + n) * 16, 16)]
                ws = [cv.at[jnp.full((16,), i, jnp.int32)].get(
                    mode="promise_in_bounds") for i in range(16)]
                rr = rows_v[r]
                nc = n * C
                for k in range(4):
                    p = [ws[c] * rr[nc + c, pl.ds(k * 16, 16)]
                         + ws[8 + c] * rr[nc + c, pl.ds(K + k * 16, 16)]
                         for c in range(C)]
                    s = [p[0] + p[1], p[2] + p[3], p[4] + p[5], p[6] + p[7]]
                    out_v[r][pl.ds(n * K + k * 16, 16)] = (s[0] + s[1]) + (s[2] + s[3])
                return carry2

            lax.fori_loop(0, NB, node_body, 0)
            out_desc(r, blk).start()

            @pl.when(blk + RING < NBLK)
            def _():
                gather_desc(r, blk + RING).start()
        return carry

    lax.fori_loop(0, NBLK // RING, group_body, 0)
    for r in range(RING):
        out_desc(r, NBLK - RING + r).wait()


def _mid_kernel(t_ref, mix_ref, b1_ref, w2_ref, t2_ref, rl2_ref):
    t = pl.program_id(1)
    pre = t_ref[0] + mix_ref[0] + b1_ref[...]
    x = jnp.tanh(pre)
    y = jnp.dot(x, w2_ref[...], preferred_element_type=jnp.float32)
    t2_ref[0] = y[:, :K]
    grow = lax.broadcasted_iota(jnp.int32, (NT, 1), 0) + t * NT
    rl2_ref[0] = y[:, K:] * (grow != 0).astype(jnp.float32)


def _mid_call(t1, mix1, b1, w2):
    return pl.pallas_call(
        _mid_kernel,
        grid=(BG, NCH),
        in_specs=[
            pl.BlockSpec((1, NT, K), lambda b, t: (b, t, 0)),
            pl.BlockSpec((1, NT, K), lambda b, t: (b, t, 0)),
            pl.BlockSpec((1, K), lambda b, t: (0, 0)),
            pl.BlockSpec((K, 3 * K), lambda b, t: (0, 0)),
        ],
        out_specs=[
            pl.BlockSpec((1, NT, K), lambda b, t: (b, t, 0)),
            pl.BlockSpec((1, NT, 2 * K), lambda b, t: (b, t, 0)),
        ],
        out_shape=[
            jax.ShapeDtypeStruct((BG, N, K), jnp.float32),
            jax.ShapeDtypeStruct((BG, N, 2 * K), jnp.float32),
        ],
    )(t1, mix1, b1, w2)


def _fin_kernel(t_ref, mix_ref, b2_ref, wc_ref, bc_ref, out_ref, acc_ref):
    t = pl.program_id(1)
    pre = t_ref[0] + mix_ref[0] + b2_ref[...]
    x = jnp.tanh(pre)
    part = jnp.max(x, axis=0, keepdims=True)

    @pl.when(t == 0)
    def _():
        acc_ref[...] = part

    @pl.when(t > 0)
    def _():
        acc_ref[...] = jnp.maximum(acc_ref[...], part)

    @pl.when(t == NCH - 1)
    def _():
        logits = jnp.dot(acc_ref[...], wc_ref[...],
                         preferred_element_type=jnp.float32) + bc_ref[...]
        mx = jnp.max(logits, axis=1, keepdims=True)
        e = jnp.exp(logits - mx)
        out_ref[0] = e / jnp.sum(e, axis=1, keepdims=True)


def _fin_call(t2, mix2, b2, w_cls, b_cls):
    return pl.pallas_call(
        _fin_kernel,
        grid=(BG, NCH),
        in_specs=[
            pl.BlockSpec((1, NT, K), lambda b, t: (b, t, 0)),
            pl.BlockSpec((1, NT, K), lambda b, t: (b, t, 0)),
            pl.BlockSpec((1, K), lambda b, t: (0, 0)),
            pl.BlockSpec((K, CLS), lambda b, t: (0, 0)),
            pl.BlockSpec((1, CLS), lambda b, t: (0, 0)),
        ],
        out_specs=pl.BlockSpec((1, 1, CLS), lambda b, t: (b, 0, 0)),
        out_shape=jax.ShapeDtypeStruct((BG, 1, CLS), jnp.float32),
        scratch_shapes=[pltpu.VMEM((1, K), jnp.float32)],
    )(t2, mix2, b2, w_cls, b_cls)


def kernel(nodes, children, w_t1, w_l1, w_r1, b1, w_t2, w_l2, w_r2, b2, w_cls, b_cls):
    ch = children.astype(jnp.int32)
    w1 = jnp.concatenate([w_t1, w_r1, w_l1], axis=1)
    w2 = jnp.concatenate([w_t2, w_r2, w_l2], axis=1)

    outs = []
    for g in range(G):
        t1, rl1, coef, gidx = _prep_call(nodes, ch, w1, g * BG)
        gidx_flat = gidx.reshape(NODES * C)
        coef_flat = coef.reshape(NODES * 16)
        mix1 = _sc_mix(rl1.reshape(NODES, 2 * K), gidx_flat, coef_flat)
        t2, rl2 = _mid_call(t1, mix1.reshape(BG, N, K), b1.reshape(1, K), w2)
        mix2 = _sc_mix(rl2.reshape(NODES, 2 * K), gidx_flat, coef_flat)
        outs.append(_fin_call(t2, mix2.reshape(BG, N, K), b2.reshape(1, K),
                              w_cls, b_cls.reshape(1, CLS)).reshape(BG, CLS))
    return jnp.concatenate(outs, axis=0)


# speedup vs baseline: 1.2785x; 1.0848x over previous
import functools

import jax
import jax.numpy as jnp
from jax import lax
from jax.experimental import pallas as pl
from jax.experimental.pallas import tpu as pltpu
from jax.experimental.pallas import tpu_sc as plsc

B, N, C = 8, 5000, 8
F = 128
K = 64
CLS = 104

G = 2
BG = B // G

NC, NS = 2, 16
NW = NC * NS
NODES = BG * N
NPW = NODES // NW
NB = 5
NBLK = NPW // NB


NCH = 1
NT = N // NCH


def _prep_kernel(nodes_ref, ch_ref, w_ref, t_ref, rl_ref, coef_ref, gidx_ref):
    b = pl.program_id(0)
    t = pl.program_id(1)
    x = nodes_ref[0]
    ch = ch_ref[0]
    y = jnp.dot(x, w_ref[...], preferred_element_type=jnp.float32)
    t_ref[0] = y[:, :K]
    grow = lax.broadcasted_iota(jnp.int32, (NT, 1), 0) + t * NT
    rl_ref[0] = y[:, K:] * (grow != 0).astype(jnp.float32)

    cf = ch.astype(jnp.float32)
    m = jnp.minimum(cf, 1.0)
    nsib = jnp.sum((cf != 0.0).astype(jnp.float32), axis=1, keepdims=True)
    jj = lax.broadcasted_iota(jnp.int32, (NT, C), 1).astype(jnp.float32)
    denom = jnp.where(nsib == 1.0, 1.0, nsib - 1.0)
    denom = jnp.where(denom == 0.0, 1.0, denom)
    singles = jnp.where(jj == 0.0, 0.5, 0.0) + jnp.zeros_like(m)
    cr = jnp.where(nsib == 1.0, singles, jj * m / denom)
    cl = (1.0 - cr) * m
    coef_ref[0] = jnp.concatenate([cr, cl], axis=1)
    gidx_ref[0] = ch + b * N


def _prep_call(nodes, ch, w1, goff):
    return pl.pallas_call(
        _prep_kernel,
        grid=(BG, NCH),
        in_specs=[
            pl.BlockSpec((1, NT, F), lambda b, t: (b + goff, t, 0)),
            pl.BlockSpec((1, NT, C), lambda b, t: (b + goff, t, 0)),
            pl.BlockSpec((F, 3 * K), lambda b, t: (0, 0)),
        ],
        out_specs=[
            pl.BlockSpec((1, NT, K), lambda b, t: (b, t, 0)),
            pl.BlockSpec((1, NT, 2 * K), lambda b, t: (b, t, 0)),
            pl.BlockSpec((1, NT, 16), lambda b, t: (b, t, 0)),
            pl.BlockSpec((1, NT, C), lambda b, t: (b, t, 0)),
        ],
        out_shape=[
            jax.ShapeDtypeStruct((BG, N, K), jnp.float32),
            jax.ShapeDtypeStruct((BG, N, 2 * K), jnp.float32),
            jax.ShapeDtypeStruct((BG, N, 16), jnp.float32),
            jax.ShapeDtypeStruct((BG, N, C), jnp.int32),
        ],
    )(nodes, ch, w1)


_sc_mesh = plsc.VectorSubcoreMesh(core_axis_name="c", subcore_axis_name="s")


RING = 5


@functools.partial(
    pl.kernel,
    out_type=jax.ShapeDtypeStruct((NODES * K,), jnp.float32),
    mesh=_sc_mesh,
    scratch_types=[
        pltpu.VMEM((NPW * C,), jnp.int32),
        pltpu.VMEM((NPW * 16,), jnp.float32),
    ] + [pltpu.VMEM((NB * C, 2 * K), jnp.float32)] * RING
      + [pltpu.VMEM((NB * K,), jnp.float32)] * RING
      + [pltpu.SemaphoreType.DMA] * (2 * RING),
)
def _sc_mix(rl_hbm, gidx_hbm, coef_hbm, out_hbm, idx_all, coef_all, *rest):
    rows_v = rest[:RING]
    out_v = rest[RING:2 * RING]
    gsem = rest[2 * RING:3 * RING]
    osem = rest[3 * RING:]
    wid = lax.axis_index("s") * NC + lax.axis_index("c")
    wbase = wid * NPW

    pltpu.sync_copy(gidx_hbm.at[pl.ds(wbase * C, NPW * C)], idx_all)

    def gather_desc(slot, blk):
        isl = idx_all.at[pl.ds(blk * NB * C, NB * C)]
        return pltpu.make_async_copy(rl_hbm.at[isl], rows_v[slot], gsem[slot])

    def out_desc(slot, blk):
        dst = out_hbm.at[pl.ds((wbase + blk * NB) * K, NB * K)]
        return pltpu.make_async_copy(out_v[slot], dst, osem[slot])

    for r in range(RING):
        gather_desc(r, r).start()
    pltpu.sync_copy(coef_hbm.at[pl.ds(wbase * 16, NPW * 16)], coef_all)

    def group_body(gi, carry):
        for r in range(RING):
            blk = gi * RING + r
            gather_desc(r, blk).wait()

            @pl.when(gi > 0)
            def _():
                out_desc(r, blk - RING).wait()

            @plsc.parallel_loop(0, NB, 1)
            def node_body(n):
                cv = coef_all[pl.ds((blk * NB + n) * 16, 16)]
                ws = [cv.at[jnp.full((16,), i, jnp.int32)].get(
                    mode="promise_in_bounds") for i in range(16)]
                rr = rows_v[r]
                nc = n * C
                for k in range(4):
                    p = [ws[c] * rr[nc + c, pl.ds(k * 16, 16)]
                         + ws[8 + c] * rr[nc + c, pl.ds(K + k * 16, 16)]
                         for c in range(C)]
                    s = [p[0] + p[1], p[2] + p[3], p[4] + p[5], p[6] + p[7]]
                    out_v[r][pl.ds(n * K + k * 16, 16)] = (s[0] + s[1]) + (s[2] + s[3])

            out_desc(r, blk).start()

            @pl.when(blk + RING < NBLK)
            def _():
                gather_desc(r, blk + RING).start()
        return carry

    lax.fori_loop(0, NBLK // RING, group_body, 0)
    for r in range(RING):
        out_desc(r, NBLK - RING + r).wait()


def _mid_kernel(t_ref, mix_ref, b1_ref, w2_ref, t2_ref, rl2_ref):
    t = pl.program_id(1)
    pre = t_ref[0] + mix_ref[0] + b1_ref[...]
    x = jnp.tanh(pre)
    y = jnp.dot(x, w2_ref[...], preferred_element_type=jnp.float32)
    t2_ref[0] = y[:, :K]
    grow = lax.broadcasted_iota(jnp.int32, (NT, 1), 0) + t * NT
    rl2_ref[0] = y[:, K:] * (grow != 0).astype(jnp.float32)


def _mid_call(t1, mix1, b1, w2):
    return pl.pallas_call(
        _mid_kernel,
        grid=(BG, NCH),
        in_specs=[
            pl.BlockSpec((1, NT, K), lambda b, t: (b, t, 0)),
            pl.BlockSpec((1, NT, K), lambda b, t: (b, t, 0)),
            pl.BlockSpec((1, K), lambda b, t: (0, 0)),
            pl.BlockSpec((K, 3 * K), lambda b, t: (0, 0)),
        ],
        out_specs=[
            pl.BlockSpec((1, NT, K), lambda b, t: (b, t, 0)),
            pl.BlockSpec((1, NT, 2 * K), lambda b, t: (b, t, 0)),
        ],
        out_shape=[
            jax.ShapeDtypeStruct((BG, N, K), jnp.float32),
            jax.ShapeDtypeStruct((BG, N, 2 * K), jnp.float32),
        ],
    )(t1, mix1, b1, w2)


def _fin_kernel(t_ref, mix_ref, b2_ref, wc_ref, bc_ref, out_ref, acc_ref):
    t = pl.program_id(1)
    pre = t_ref[0] + mix_ref[0] + b2_ref[...]
    x = jnp.tanh(pre)
    part = jnp.max(x, axis=0, keepdims=True)

    @pl.when(t == 0)
    def _():
        acc_ref[...] = part

    @pl.when(t > 0)
    def _():
        acc_ref[...] = jnp.maximum(acc_ref[...], part)

    @pl.when(t == NCH - 1)
    def _():
        logits = jnp.dot(acc_ref[...], wc_ref[...],
                         preferred_element_type=jnp.float32) + bc_ref[...]
        mx = jnp.max(logits, axis=1, keepdims=True)
        e = jnp.exp(logits - mx)
        out_ref[0] = e / jnp.sum(e, axis=1, keepdims=True)


def _fin_call(t2, mix2, b2, w_cls, b_cls):
    return pl.pallas_call(
        _fin_kernel,
        grid=(BG, NCH),
        in_specs=[
            pl.BlockSpec((1, NT, K), lambda b, t: (b, t, 0)),
            pl.BlockSpec((1, NT, K), lambda b, t: (b, t, 0)),
            pl.BlockSpec((1, K), lambda b, t: (0, 0)),
            pl.BlockSpec((K, CLS), lambda b, t: (0, 0)),
            pl.BlockSpec((1, CLS), lambda b, t: (0, 0)),
        ],
        out_specs=pl.BlockSpec((1, 1, CLS), lambda b, t: (b, 0, 0)),
        out_shape=jax.ShapeDtypeStruct((BG, 1, CLS), jnp.float32),
        scratch_shapes=[pltpu.VMEM((1, K), jnp.float32)],
    )(t2, mix2, b2, w_cls, b_cls)


def kernel(nodes, children, w_t1, w_l1, w_r1, b1, w_t2, w_l2, w_r2, b2, w_cls, b_cls):
    ch = children.astype(jnp.int32)
    w1 = jnp.concatenate([w_t1, w_r1, w_l1], axis=1)
    w2 = jnp.concatenate([w_t2, w_r2, w_l2], axis=1)

    outs = []
    for g in range(G):
        t1, rl1, coef, gidx = _prep_call(nodes, ch, w1, g * BG)
        gidx_flat = gidx.reshape(NODES * C)
        coef_flat = coef.reshape(NODES * 16)
        mix1 = _sc_mix(rl1.reshape(NODES, 2 * K), gidx_flat, coef_flat)
        t2, rl2 = _mid_call(t1, mix1.reshape(BG, N, K), b1.reshape(1, K), w2)
        mix2 = _sc_mix(rl2.reshape(NODES, 2 * K), gidx_flat, coef_flat)
        outs.append(_fin_call(t2, mix2.reshape(BG, N, K), b2.reshape(1, K),
                              w_cls, b_cls.reshape(1, CLS)).reshape(BG, CLS))
    return jnp.concatenate(outs, axis=0)
